# Initial kernel scaffold; baseline (speedup 1.0000x reference)
import jax
import jax.numpy as jnp
from jax.experimental import pallas as pl


def kernel(x, Wg, bg, W1, W2, W3):
    raise NotImplementedError("write your pallas kernel here")


# speedup vs baseline: 3.8316x; 3.8316x over previous
import jax
import jax.numpy as jnp
from jax.experimental import pallas as pl
from jax.experimental.pallas import tpu as pltpu

T = 2048
D = 1024
H = 2048
E = 8
CAP = 320
NH = 2
HB = H // NH
TB = 128
NTB = T // TB


def _moe_body(x_ref, wg_ref, bg_ref, w1_ref, w2_ref, w3_ref, out_ref,
              route_ref, lbl_ref, pos_ref, xe_ref, acc_ref):
    e = pl.program_id(0)
    h = pl.program_id(1)

    @pl.when(jnp.logical_and(e == 0, h == 0))
    def _routing():
        xf = x_ref[...]
        logits = jnp.dot(xf, wg_ref[...],
                         preferred_element_type=jnp.float32) + bg_ref[...]
        eio = jax.lax.broadcasted_iota(jnp.int32, (T, E), 1).astype(jnp.float32)
        l1 = jnp.max(logits, axis=1, keepdims=True)
        i1 = jnp.min(jnp.where(logits == l1, eio, float(E)),
                     axis=1, keepdims=True)
        masked = jnp.where(eio == i1, -jnp.inf, logits)
        l2 = jnp.max(masked, axis=1, keepdims=True)
        i2 = jnp.min(jnp.where(masked == l2, eio, float(E)),
                     axis=1, keepdims=True)
        lbl_ref[...] = ((eio == i1) | (eio == i2)).astype(jnp.float32)

        r = jax.lax.broadcasted_iota(jnp.int32, (TB, TB), 0)
        c = jax.lax.broadcasted_iota(jnp.int32, (TB, TB), 1)
        tri = (r >= c).astype(jnp.float32)

        def body(b, carry):
            blk = lbl_ref[pl.ds(b * TB, TB), :]
            s = jnp.dot(tri, blk, preferred_element_type=jnp.float32) + carry
            pos_ref[pl.ds(b * TB, TB), :] = s
            return s[TB - 1:TB, :]

        jax.lax.fori_loop(0, NTB, body, jnp.zeros((1, E), jnp.float32))

        pos = pos_ref[...]
        pos1 = jnp.sum(pos * (eio == i1), axis=1, keepdims=True)
        pos2 = jnp.sum(pos * (eio == i2), axis=1, keepdims=True)
        v1 = (pos1 <= float(CAP)).astype(jnp.float32)
        v2 = (pos2 <= float(CAP)).astype(jnp.float32)
        sentinel = float(E * CAP)
        col0 = jnp.where(v1 > 0.0, i1 * CAP + pos1 - 1.0, sentinel)
        col1 = jnp.where(v2 > 0.0, i2 * CAP + pos2 - 1.0, sentinel)
        e2 = jnp.exp(l2 - l1)
        den = 1.0 + e2
        cw0 = v1 / den
        cw1 = v2 * e2 / den
        route_ref[...] = jnp.concatenate(
            [col0, col1, cw0, cw1, jnp.zeros((T, 4), jnp.float32)], axis=1)

    col0 = route_ref[:, 0:1]
    col1 = route_ref[:, 1:2]
    li = (jax.lax.broadcasted_iota(jnp.int32, (1, CAP), 1).astype(jnp.float32)
          + (e * CAP).astype(jnp.float32))

    @pl.when(h == 0)
    def _dispatch():
        pe = ((col0 == li) | (col1 == li)).astype(jnp.float32)
        xe_ref[...] = jax.lax.dot_general(
            pe, x_ref[...], (((0,), (0,)), ((), ())),
            preferred_element_type=jnp.float32)

    xe = xe_ref[...]
    hp = jnp.dot(xe, w1_ref[0], preferred_element_type=jnp.float32)
    gp = jnp.dot(xe, w2_ref[0], preferred_element_type=jnp.float32)
    act = hp * (1.0 / (1.0 + jnp.exp(-hp))) * gp
    yb = jnp.dot(act, w3_ref[0], preferred_element_type=jnp.float32)

    @pl.when(h == 0)
    def _init_acc():
        acc_ref[...] = yb

    @pl.when(h > 0)
    def _add_acc():
        acc_ref[...] += yb

    @pl.when(h == NH - 1)
    def _combine():
        cw0 = route_ref[:, 2:3]
        cw1 = route_ref[:, 3:4]
        ce = (col0 == li).astype(jnp.float32) * cw0 \
            + (col1 == li).astype(jnp.float32) * cw1
        ob = jnp.dot(ce, acc_ref[...], preferred_element_type=jnp.float32)

        @pl.when(e == 0)
        def _():
            out_ref[...] = ob

        @pl.when(e > 0)
        def _():
            out_ref[...] += ob


def kernel(x, Wg, bg, W1, W2, W3):
    b, s, d = x.shape
    xf = x.reshape(s, d)
    bg2 = bg.reshape(1, E)
    out = pl.pallas_call(
        _moe_body,
        grid=(E, NH),
        in_specs=[
            pl.BlockSpec((T, D), lambda e, h: (0, 0)),
            pl.BlockSpec((D, E), lambda e, h: (0, 0)),
            pl.BlockSpec((1, E), lambda e, h: (0, 0)),
            pl.BlockSpec((1, D, HB), lambda e, h: (e, 0, h)),
            pl.BlockSpec((1, D, HB), lambda e, h: (e, 0, h)),
            pl.BlockSpec((1, HB, D), lambda e, h: (e, h, 0)),
        ],
        out_specs=pl.BlockSpec((T, D), lambda e, h: (0, 0)),
        out_shape=jax.ShapeDtypeStruct((T, D), jnp.float32),
        scratch_shapes=[
            pltpu.VMEM((T, 8), jnp.float32),
            pltpu.VMEM((T, E), jnp.float32),
            pltpu.VMEM((T, E), jnp.float32),
            pltpu.VMEM((CAP, D), jnp.float32),
            pltpu.VMEM((CAP, D), jnp.float32),
        ],
        compiler_params=pltpu.CompilerParams(
            dimension_semantics=("arbitrary", "arbitrary")),
    )(xf, Wg, bg2, W1, W2, W3)
    return out.reshape(b, s, d)
